# baseline (speedup 1.0000x reference)
import jax
import jax.numpy as jnp
from jax import lax
from jax.experimental import pallas as pl
from jax.experimental.pallas import tpu as pltpu
from jax.experimental.pallas import tpu_sc as plsc

NUM_CORES = 2
NUM_SUBCORES = 16
NUM_WORKERS = NUM_CORES * NUM_SUBCORES
LANES = 16

B = 4096 * 200
D = 64
G = 128
GPI = 8
ROWS_PER_ITER = GPI * G
ROWS_PER_WORKER = B // NUM_WORKERS
NITERS = ROWS_PER_WORKER // ROWS_PER_ITER
OUT_SCALE = 8.0


def _emb_body(ids_hbm, table_hbm, out_hbm, idx_v, pidx_v, wide_v, out_v,
              sem_g, sem_w):
    wid = lax.axis_index("s") * NUM_CORES + lax.axis_index("c")
    grp_base = wid * (ROWS_PER_WORKER // G)
    out_base = wid * (ROWS_PER_WORKER // 2)

    def gather(j, jb):
        return pltpu.async_copy(
            table_hbm.at[pidx_v.at[j]], wide_v.at[jb], sem_g)

    def write(bb, o):
        pltpu.async_copy(out_v.at[bb], out_hbm.at[pl.ds(o, G)], sem_w)

    def wait_write(bb, o):
        pltpu.make_async_copy(
            out_v.at[bb], out_hbm.at[pl.ds(o, G)], sem_w).wait()

    def iter_body(it, carry):
        g0 = pl.multiple_of(grp_base + it * GPI, GPI)
        o0 = pl.multiple_of(out_base + it * (ROWS_PER_ITER // 2), 8)
        pltpu.sync_copy(ids_hbm.at[pl.ds(g0, GPI)], idx_v)
        for j in range(GPI):
            for v in range(G // LANES):
                sl = (j, pl.ds(v * LANES, LANES))
                pidx_v[sl] = lax.shift_right_logical(idx_v[sl], 1)
        handles = [None] * GPI
        handles[0] = gather(0, 0)
        for j in range(GPI):
            if j < GPI - 1:
                handles[j + 1] = gather(j + 1, (j + 1) % 2)
            bb = (j // 2) % 2
            if j % 2 == 0 and j // 2 >= 2:
                wait_write(bb, o0 + (j // 2 - 2) * G)
            handles[j].wait()

            def sel_block(blk, carry2):
                parv = idx_v[j, pl.ds(blk * LANES, LANES)] & 1
                for k in range(LANES):
                    src = parv[k] * D
                    row = (j % 2) * (G // 2) + blk * (LANES // 2) + k // 2
                    dst = (k % 2) * D
                    for v in range(D // LANES):
                        out_v[bb, row, pl.ds(dst + v * LANES, LANES)] = (
                            wide_v[j % 2, blk * LANES + k,
                                   pl.ds(src + v * LANES, LANES)] * OUT_SCALE
                        )
                return carry2

            lax.fori_loop(0, G // LANES, sel_block, 0)
            if j % 2 == 1:
                write(bb, o0 + (j // 2) * G)
        return carry

    def iter_with_drain(it, carry):
        @pl.when(it > 0)
        def _():
            o_prev = pl.multiple_of(
                out_base + (it - 1) * (ROWS_PER_ITER // 2), 8)
            wait_write(0, o_prev + 2 * G)
            wait_write(1, o_prev + 3 * G)
        return iter_body(it, carry)

    lax.fori_loop(0, NITERS, iter_with_drain, 0)
    o_last = pl.multiple_of(out_base + (NITERS - 1) * (ROWS_PER_ITER // 2), 8)
    wait_write(0, o_last + 2 * G)
    wait_write(1, o_last + 3 * G)


@jax.jit
def _emb(ids_grouped, table128):
    mesh = plsc.VectorSubcoreMesh(
        core_axis_name="c", subcore_axis_name="s",
        num_cores=NUM_CORES, num_subcores=NUM_SUBCORES,
    )
    return pl.kernel(
        _emb_body,
        out_type=jax.ShapeDtypeStruct((B // 2, 2 * D), jnp.float32),
        mesh=mesh,
        scratch_types=[
            pltpu.VMEM((GPI, G), jnp.int32),
            pltpu.VMEM((GPI, G), jnp.int32),
            pltpu.VMEM((2, G, 2 * D), jnp.float32),
            pltpu.VMEM((2, G, 2 * D), jnp.float32),
            pltpu.SemaphoreType.DMA,
            pltpu.SemaphoreType.DMA,
        ],
    )(ids_grouped, table128)


def kernel(input_ids, weight):
    ids = input_ids.astype(jnp.int32).reshape(B // G, G)
    table128 = weight.reshape(500000, 128)
    out = _emb(ids, table128)
    return out.reshape(input_ids.shape + (D,))


# speedup vs baseline: 1.3384x; 1.3384x over previous
import jax
import jax.numpy as jnp
from jax import lax
from jax.experimental import pallas as pl
from jax.experimental.pallas import tpu as pltpu
from jax.experimental.pallas import tpu_sc as plsc

NUM_CORES = 2
NUM_SUBCORES = 16
NUM_WORKERS = NUM_CORES * NUM_SUBCORES
LANES = 16

NB = 4096
NT = 200
B = NB * NT
D = 64
ROWS_PER_WORKER = B // NUM_WORKERS
B_PER_WORKER = NB // NUM_WORKERS
SLAB_B = 2
SLAB = SLAB_B * NT
NSLAB = ROWS_PER_WORKER // SLAB
NBUF = 4
SPLITS = ((0, 0, 0, 128), (0, 128, 128, 72),
          (1, 0, 200, 128), (1, 128, 328, 72))
OUT_SCALE = 8.0


def _emb_body(ids_hbm, tab_hbm, out_hbm, idx_v, rows_v, *sems):
    sem_g, sem_w = sems[:NBUF], sems[NBUF:]
    wid = lax.axis_index("s") * NUM_CORES + lax.axis_index("c")
    i_base = wid * ROWS_PER_WORKER
    b_base = wid * B_PER_WORKER

    pltpu.sync_copy(ids_hbm.at[pl.ds(i_base, ROWS_PER_WORKER)], idx_v)

    def fire_gathers(s, q):
        for db, doff, ioff, n in SPLITS:
            pltpu.async_copy(
                tab_hbm.at[idx_v.at[pl.ds(s * SLAB + ioff, n)]],
                rows_v.at[q, db, pl.ds(doff, n)], sem_g[q])

    def wait_gathers(s, q):
        for db, doff, ioff, n in SPLITS:
            pltpu.make_async_copy(
                tab_hbm.at[idx_v.at[pl.ds(s * SLAB + ioff, n)]],
                rows_v.at[q, db, pl.ds(doff, n)], sem_g[q]).wait()

    def fire_write(s, q):
        pltpu.async_copy(
            rows_v.at[q], out_hbm.at[pl.ds(b_base + s * SLAB_B, SLAB_B)],
            sem_w[q])

    def wait_write(s, q):
        pltpu.make_async_copy(
            rows_v.at[q], out_hbm.at[pl.ds(b_base + s * SLAB_B, SLAB_B)],
            sem_w[q]).wait()

    def scale_slab(q):
        def row(r, carry):
            for db in range(SLAB_B):
                for c in range(D // LANES):
                    sl = (q, db, r, pl.ds(c * LANES, LANES))
                    rows_v[sl] = rows_v[sl] * OUT_SCALE
            return carry
        lax.fori_loop(0, NT, row, 0, unroll=8)

    fire_gathers(0, 0)

    def pair_body(p, carry):
        for q in range(NBUF):
            s = p * NBUF + q
            qn = (q + 1) % NBUF
            if q == NBUF - 1:
                @pl.when(p < NSLAB // NBUF - 1)
                def _():
                    wait_write(s - (NBUF - 1), qn)
                    fire_gathers(s + 1, qn)
            else:
                @pl.when(p * NBUF + q >= NBUF - 1)
                def _():
                    wait_write(s - (NBUF - 1), qn)
                fire_gathers(s + 1, qn)
            wait_gathers(s, q)
            scale_slab(q)
            fire_write(s, q)
        return carry

    lax.fori_loop(0, NSLAB // NBUF, pair_body, 0)
    for s in range(NSLAB - NBUF, NSLAB):
        wait_write(s, s % NBUF)


@jax.jit
def _emb(ids_flat, weight):
    mesh = plsc.VectorSubcoreMesh(
        core_axis_name="c", subcore_axis_name="s",
        num_cores=NUM_CORES, num_subcores=NUM_SUBCORES,
    )
    return pl.kernel(
        _emb_body,
        out_type=jax.ShapeDtypeStruct((NB, NT, D), jnp.float32),
        mesh=mesh,
        scratch_types=[
            pltpu.VMEM((ROWS_PER_WORKER,), jnp.int32),
            pltpu.VMEM((NBUF, SLAB_B, NT, D), jnp.float32),
        ] + [pltpu.SemaphoreType.DMA] * (2 * NBUF),
        compiler_params=pltpu.CompilerParams(use_tc_tiling_on_sc=False),
    )(ids_flat, weight)


def kernel(input_ids, weight):
    ids = input_ids.astype(jnp.int32).reshape(-1)
    return _emb(ids, weight)
